# Initial kernel scaffold; baseline (speedup 1.0000x reference)
import jax
import jax.numpy as jnp
from jax.experimental import pallas as pl


def kernel(tokens, table):
    raise NotImplementedError("write your pallas kernel here")


# speedup vs baseline: 135.7277x; 135.7277x over previous
import functools

import jax
import jax.numpy as jnp
from jax import lax
from jax.experimental import pallas as pl
from jax.experimental.pallas import tpu as pltpu
from jax.experimental.pallas import tpu_sc as plsc

BATCH = 16384
SEQ = 200
B = BATCH * SEQ
NC = 2
NS = 16
NW = NC * NS
BPW = B // NW
CHUNK = 12800
NSTEP = BPW // CHUNK


def _gather_kernel(idx_hbm, table_hbm, out_hbm, idx_v, val_v, sem):
    wid = lax.axis_index("s") * NC + lax.axis_index("c")
    base = wid * BPW
    for i in range(NSTEP):
        off = base + i * CHUNK
        pltpu.sync_copy(idx_hbm.at[pl.ds(off, CHUNK)], idx_v)
        pltpu.async_copy(table_hbm.at[idx_v], val_v, sem).wait()
        pltpu.sync_copy(val_v, out_hbm.at[pl.ds(off, CHUNK)])


def kernel(tokens, table):
    idx = tokens.reshape(B).astype(jnp.int32)
    mesh = plsc.VectorSubcoreMesh(core_axis_name="c", subcore_axis_name="s")
    run = functools.partial(
        pl.kernel,
        mesh=mesh,
        out_type=jax.ShapeDtypeStruct((B,), jnp.float32),
        scratch_types=[
            pltpu.VMEM((CHUNK,), jnp.int32),
            pltpu.VMEM((CHUNK,), jnp.float32),
            pltpu.SemaphoreType.DMA,
        ],
    )(_gather_kernel)
    out = run(idx, table)
    return out.reshape(BATCH, SEQ)


# speedup vs baseline: 140.9246x; 1.0383x over previous
import functools

import jax
import jax.numpy as jnp
from jax import lax
from jax.experimental import pallas as pl
from jax.experimental.pallas import tpu as pltpu
from jax.experimental.pallas import tpu_sc as plsc

BATCH = 16384
SEQ = 200
B = BATCH * SEQ
NC = 2
NS = 16
NW = NC * NS
BPW = B // NW
CHUNK = 12800
NSTEP = BPW // CHUNK


def _gather_kernel(idx_hbm, table_hbm, out_hbm, idx_v0, idx_v1, val_v0,
                   val_v1, isem0, isem1, gsem0, gsem1, ssem0, ssem1):
    idx_v = (idx_v0, idx_v1)
    val_v = (val_v0, val_v1)
    isem = (isem0, isem1)
    gsem = (gsem0, gsem1)
    ssem = (ssem0, ssem1)
    wid = lax.axis_index("s") * NC + lax.axis_index("c")
    base = wid * BPW

    def idx_copy(i):
        return pltpu.async_copy(
            idx_hbm.at[pl.ds(base + i * CHUNK, CHUNK)], idx_v[i % 2],
            isem[i % 2])

    def gather(i):
        return pltpu.async_copy(
            table_hbm.at[idx_v[i % 2]], val_v[i % 2], gsem[i % 2])

    def store(i):
        return pltpu.async_copy(
            val_v[i % 2], out_hbm.at[pl.ds(base + i * CHUNK, CHUNK)],
            ssem[i % 2])

    ic, gc, st = {}, {}, {}
    ic[0] = idx_copy(0)
    ic[1] = idx_copy(1)
    ic[0].wait()
    gc[0] = gather(0)
    for i in range(NSTEP):
        if i + 1 < NSTEP:
            ic[i + 1].wait()
            if i - 1 >= 0:
                st[i - 1].wait()
            gc[i + 1] = gather(i + 1)
        gc[i].wait()
        if i + 2 < NSTEP:
            ic[i + 2] = idx_copy(i + 2)
        st[i] = store(i)
    st[NSTEP - 2].wait()
    st[NSTEP - 1].wait()


def kernel(tokens, table):
    idx = tokens.reshape(B).astype(jnp.int32)
    mesh = plsc.VectorSubcoreMesh(core_axis_name="c", subcore_axis_name="s")
    run = functools.partial(
        pl.kernel,
        mesh=mesh,
        out_type=jax.ShapeDtypeStruct((B,), jnp.float32),
        scratch_types=[
            pltpu.VMEM((CHUNK,), jnp.int32),
            pltpu.VMEM((CHUNK,), jnp.int32),
            pltpu.VMEM((CHUNK,), jnp.float32),
            pltpu.VMEM((CHUNK,), jnp.float32),
            pltpu.SemaphoreType.DMA,
            pltpu.SemaphoreType.DMA,
            pltpu.SemaphoreType.DMA,
            pltpu.SemaphoreType.DMA,
            pltpu.SemaphoreType.DMA,
            pltpu.SemaphoreType.DMA,
        ],
    )(_gather_kernel)
    out = run(idx, table)
    return out.reshape(BATCH, SEQ)


# speedup vs baseline: 218.1654x; 1.5481x over previous
import functools

import jax
import jax.numpy as jnp
from jax import lax
from jax.experimental import pallas as pl
from jax.experimental.pallas import tpu as pltpu
from jax.experimental.pallas import tpu_sc as plsc

BATCH = 16384
SEQ = 200
VOCAB = 1000000
B = BATCH * SEQ
NC = 2
NS = 16
NW = NC * NS
BPW = B // NW
CHUNK = 12800
NSTEP = BPW // CHUNK
STAGE_TILES = 8
STAGE_W = VOCAB // STAGE_TILES
STAGE_CHUNK = 5000
STAGE_ROUNDS = STAGE_W // STAGE_CHUNK


def _gather_kernel(idx_hbm, table_hbm, out_hbm, tab_s, idx_v0, idx_v1,
                   val_v0, val_v1, isem0, isem1, gsem0, gsem1, ssem0, ssem1):
    idx_v = (idx_v0, idx_v1)
    val_v = (val_v0, val_v1)
    isem = (isem0, isem1)
    gsem = (gsem0, gsem1)
    ssem = (ssem0, ssem1)
    sid = lax.axis_index("s")
    wid = sid * NC + lax.axis_index("c")
    base = wid * BPW

    @pl.when(sid < STAGE_TILES)
    def _stage():
        tbase = sid * STAGE_W
        ld = {}
        ld[0] = pltpu.async_copy(
            table_hbm.at[pl.ds(tbase, STAGE_CHUNK)],
            val_v0.at[pl.ds(0, STAGE_CHUNK)], gsem0)
        for j in range(STAGE_ROUNDS):
            if j + 1 < STAGE_ROUNDS:
                ld[j + 1] = pltpu.async_copy(
                    table_hbm.at[pl.ds(tbase + (j + 1) * STAGE_CHUNK,
                                       STAGE_CHUNK)],
                    val_v[(j + 1) % 2].at[pl.ds(0, STAGE_CHUNK)],
                    gsem[(j + 1) % 2])
            ld[j].wait()
            pltpu.sync_copy(val_v[j % 2].at[pl.ds(0, STAGE_CHUNK)],
                            tab_s.at[pl.ds(tbase + j * STAGE_CHUNK,
                                           STAGE_CHUNK)])

    plsc.subcore_barrier()

    def idx_copy(i):
        return pltpu.async_copy(
            idx_hbm.at[pl.ds(base + i * CHUNK, CHUNK)], idx_v[i % 2],
            isem[i % 2])

    def gather(i):
        return pltpu.async_copy(
            tab_s.at[idx_v[i % 2]], val_v[i % 2], gsem[i % 2])

    def store(i):
        return pltpu.async_copy(
            val_v[i % 2], out_hbm.at[pl.ds(base + i * CHUNK, CHUNK)],
            ssem[i % 2])

    ic, gc, st = {}, {}, {}
    ic[0] = idx_copy(0)
    ic[1] = idx_copy(1)
    ic[0].wait()
    gc[0] = gather(0)
    for i in range(NSTEP):
        if i + 1 < NSTEP:
            ic[i + 1].wait()
            if i - 1 >= 0:
                st[i - 1].wait()
            gc[i + 1] = gather(i + 1)
        gc[i].wait()
        if i + 2 < NSTEP:
            ic[i + 2] = idx_copy(i + 2)
        st[i] = store(i)
    st[NSTEP - 2].wait()
    st[NSTEP - 1].wait()


def kernel(tokens, table):
    idx = tokens.reshape(B).astype(jnp.int32)
    mesh = plsc.VectorSubcoreMesh(core_axis_name="c", subcore_axis_name="s")
    run = functools.partial(
        pl.kernel,
        mesh=mesh,
        out_type=jax.ShapeDtypeStruct((B,), jnp.float32),
        scratch_types=[
            pltpu.VMEM_SHARED((VOCAB,), jnp.float32),
            pltpu.VMEM((CHUNK,), jnp.int32),
            pltpu.VMEM((CHUNK,), jnp.int32),
            pltpu.VMEM((CHUNK,), jnp.float32),
            pltpu.VMEM((CHUNK,), jnp.float32),
            pltpu.SemaphoreType.DMA,
            pltpu.SemaphoreType.DMA,
            pltpu.SemaphoreType.DMA,
            pltpu.SemaphoreType.DMA,
            pltpu.SemaphoreType.DMA,
            pltpu.SemaphoreType.DMA,
        ],
    )(_gather_kernel)
    out = run(idx, table)
    return out.reshape(BATCH, SEQ)
